# baseline (speedup 1.0000x reference)
import jax
import jax.numpy as jnp
from jax.experimental import pallas as pl

N = 10000
E = 160000
D = 256
DE = 4
SPH = 16
CEN = 4
INTER = SPH + CEN + D
K = 4
C = 16384
MINR = 0.1
NB = 4

HI = jax.lax.Precision.HIGHEST


def _q(a):
    return a


def _split3(a):
    hi = a.astype(jnp.bfloat16).astype(jnp.float32)
    lo = (a - hi).astype(jnp.bfloat16).astype(jnp.float32)
    return hi, lo


def _dot3(a, b):
    ah, al = _split3(a)
    bh, bl = _split3(b)
    return (jnp.dot(ah, bh, precision=HI) + jnp.dot(ah, bl, precision=HI)
            + jnp.dot(al, bh, precision=HI))


def kernel(x, pos, edge_index, edge_attr, batch, W_conv, W_gather):
    src, dst = edge_index[0], edge_index[1]
    P = x @ W_conv[:D, :20]
    q = edge_attr @ W_conv[D:, :20]
    out20 = jax.ops.segment_sum((P[src] + q)[::-1], dst[::-1], num_segments=N)
    A = jax.ops.segment_sum(x[src], dst, num_segments=N)
    Ea = jax.ops.segment_sum(edge_attr, dst, num_segments=N)
    feat256 = (jnp.dot(A, W_conv[:D, 20:], precision=HI)
               + jnp.dot(Ea, W_conv[D:, 20:], precision=HI))
    out = jnp.concatenate([out20, feat256], axis=-1)
    sph = out[:, :SPH]
    centers = out[:, SPH:SPH + CEN]
    feat = out[:, SPH + CEN:]
    mask = (centers[:, 0] > 0.5).astype(x.dtype)
    disp = centers[:, 1:][:, jnp.array([2, 0, 1])]
    center_pos = pos + disp
    bloom_disp = jnp.tanh(sph[:, :3 * K].reshape(N, K, 3)) * (2.0 * MINR)
    bloom_pos = (pos[:, None, :] + bloom_disp).reshape(N * K, 3)
    bloom_batch = jnp.repeat(jnp.arange(N, dtype=jnp.int32), K)
    cell = jnp.floor(bloom_pos / MINR).astype(jnp.int32)
    b_pt = batch[bloom_batch]
    h = (cell[:, 0] * 73856093) ^ (cell[:, 1] * 19349663) ^ (cell[:, 2] * 83492791) ^ (b_pt * 2654435)
    cid = jnp.mod(h, C)
    onehot = (b_pt[:, None] == jnp.arange(NB, dtype=jnp.int32)[None, :]).astype(jnp.float32)
    srows = jnp.concatenate([bloom_pos, jnp.ones((N * K, 1), jnp.float32), onehot], axis=-1)
    S = jax.ops.segment_sum(srows, cid, num_segments=C)
    cnt = S[:, 3]
    new_pos_c = S[:, :3] / jnp.clip(cnt, 1.0)[:, None]
    batc = jnp.full((C,), jnp.iinfo(jnp.int32).min, jnp.int32)
    for b in range(NB):
        batc = jnp.where(S[:, 4 + b] > 0, b, batc)
    featq = _q(feat)
    Wgq = _q(W_gather)
    F_c = jax.ops.segment_sum(featq[bloom_batch], cid, num_segments=C)
    attr_pt = _q(pos[bloom_batch] - new_pos_c[cid])
    Attr_c = jax.ops.segment_sum(attr_pt, cid, num_segments=C)
    xn1 = jnp.dot(F_c, Wgq[:D], precision=HI) + jnp.dot(Attr_c, Wgq[D:], precision=HI)
    xn2 = (jnp.dot(featq, Wgq[:D], precision=HI)
           + jnp.dot(_q(-disp), Wgq[D:], precision=HI)) * mask[:, None]
    x_new = jnp.concatenate([xn1, xn2], axis=0)
    new_pos = jnp.concatenate([new_pos_c, center_pos], axis=0)
    rep = cid.reshape(N, K)[:, 0]
    nsrc = rep[src]
    ndst = rep[dst]
    new_edge_index = jnp.stack([nsrc, ndst])
    new_edge_attr = new_pos[ndst] - new_pos[nsrc]
    new_batch = jnp.concatenate([batc, batch])
    return x_new, new_pos, new_edge_index, new_edge_attr, new_batch


# speedup vs baseline: 1.9003x; 1.9003x over previous
import functools

import jax
import jax.numpy as jnp
from jax import lax
from jax.experimental import pallas as pl
from jax.experimental.pallas import tpu as pltpu
from jax.experimental.pallas import tpu_sc as plsc

N = 10000
E = 160000
D = 256
K = 4
C = 16384
MINR = 0.1
NB = 4

NP = 10240
EP = 163840

HI = lax.Precision.HIGHEST

_MESH = plsc.VectorSubcoreMesh(core_axis_name="c", subcore_axis_name="s")


def _f32(*shape):
    return jax.ShapeDtypeStruct(shape, jnp.float32)


def _i32(*shape):
    return jax.ShapeDtypeStruct(shape, jnp.int32)


@functools.partial(
    pl.kernel,
    out_type=[_i32(EP), _i32(EP), _f32(EP, 16)],
    mesh=_MESH,
    scratch_types=[
        pltpu.VMEM((128,), jnp.int32),
        pltpu.VMEM((128,), jnp.int32),
        pltpu.VMEM((128,), jnp.int32),
        pltpu.VMEM((128,), jnp.int32),
        pltpu.VMEM((128, 128), jnp.float32),
        pltpu.VMEM((128, 128), jnp.float32),
        pltpu.VMEM((128, 16), jnp.float32),
    ],
)
def sc_edge_remap(rep_hbm, src_hbm, dst_hbm, npc_hbm,
                  ns_out, nd_out, attr_out,
                  srcb, dstb, nsb, ndb, bs, bd, bo):
    c = lax.axis_index("c")
    s = lax.axis_index("s")
    w = s * 2 + c

    def chunk(t, _):
        base = (w * 40 + t) * 128
        pltpu.sync_copy(src_hbm.at[pl.ds(base, 128)], srcb)
        pltpu.sync_copy(dst_hbm.at[pl.ds(base, 128)], dstb)
        for g in range(8):
            srcb[pl.ds(g * 16, 16)] = jnp.maximum(srcb[pl.ds(g * 16, 16)], 0)
            dstb[pl.ds(g * 16, 16)] = jnp.maximum(dstb[pl.ds(g * 16, 16)], 0)
        pltpu.sync_copy(rep_hbm.at[srcb], nsb)
        pltpu.sync_copy(rep_hbm.at[dstb], ndb)
        pltpu.sync_copy(nsb, ns_out.at[pl.ds(base, 128)])
        pltpu.sync_copy(ndb, nd_out.at[pl.ds(base, 128)])
        pltpu.sync_copy(npc_hbm.at[nsb], bs)
        pltpu.sync_copy(npc_hbm.at[ndb], bd)
        for i in range(128):
            bo[i] = bd[i, pl.ds(0, 16)] - bs[i, pl.ds(0, 16)]
        pltpu.sync_copy(bo, attr_out.at[pl.ds(base, 128)])
        return 0

    lax.fori_loop(0, 40, chunk, 0)


def _tc_feat_body(a_ref, ea_ref, o20_ref, pos_ref, wf1, wf2, wg1, wg2,
                  feat_o, xn2, bp, msc):
    feat = (jnp.dot(a_ref[...], wf1[...], precision=HI)
            + jnp.dot(ea_ref[...], wf2[...], precision=HI))
    o20 = o20_ref[...]
    pos3 = pos_ref[...][:, :3]
    n = o20.shape[0]
    sph12 = o20[:, :12]
    maskv = (o20[:, 16:17] > 0.5).astype(jnp.float32)
    dispv = jnp.concatenate([o20[:, 19:20], o20[:, 17:18], o20[:, 18:19]],
                            axis=1)
    cpos = pos3 + dispv
    post = jnp.concatenate([pos3, pos3, pos3, pos3], axis=1)
    bp12 = post + jnp.tanh(sph12) * (2.0 * MINR)
    nd16 = jnp.concatenate([-dispv, jnp.zeros((n, 13), jnp.float32)], axis=1)
    xn2v = (jnp.dot(feat, wg1[...], precision=HI)
            + jnp.dot(nd16, wg2[...], precision=HI)) * maskv
    feat_o[...] = feat
    xn2[...] = xn2v
    bp[...] = jnp.concatenate([bp12, jnp.zeros((n, 4), jnp.float32)], axis=1)
    msc[...] = jnp.concatenate([cpos, -dispv, maskv,
                                jnp.zeros((n, 9), jnp.float32)], axis=1)


def _tc_feat(A, EA16, O20, pos8, Wf1, Wf2, Wg1, Wg2):
    BR = 1000
    grid = (N // BR,)
    row = lambda i: (i, 0)
    full = lambda i: (0, 0)
    return pl.pallas_call(
        _tc_feat_body,
        grid=grid,
        in_specs=[
            pl.BlockSpec((BR, 256), row),
            pl.BlockSpec((BR, 16), row),
            pl.BlockSpec((BR, 32), row),
            pl.BlockSpec((BR, 16), row),
            pl.BlockSpec((256, 256), full),
            pl.BlockSpec((16, 256), full),
            pl.BlockSpec((256, 256), full),
            pl.BlockSpec((16, 256), full),
        ],
        out_specs=[
            pl.BlockSpec((BR, 256), row),
            pl.BlockSpec((BR, 256), row),
            pl.BlockSpec((BR, 16), row),
            pl.BlockSpec((BR, 16), row),
        ],
        out_shape=[_f32(N, 256), _f32(N, 256), _f32(N, 16), _f32(N, 16)],
    )(A, EA16, O20, pos8, Wf1, Wf2, Wg1, Wg2)


def _tc_out_body(F, atr, wg1, wg2, out):
    out[...] = (jnp.dot(F[...], wg1[...], precision=HI)
                + jnp.dot(atr[...], wg2[...], precision=HI))


def _tc_out(F, ATR16, Wg1, Wg2):
    BR = 1024
    grid = (C // BR,)
    row = lambda i: (i, 0)
    full = lambda i: (0, 0)
    return pl.pallas_call(
        _tc_out_body,
        grid=grid,
        in_specs=[
            pl.BlockSpec((BR, 256), row),
            pl.BlockSpec((BR, 16), row),
            pl.BlockSpec((256, 256), full),
            pl.BlockSpec((16, 256), full),
        ],
        out_specs=[pl.BlockSpec((BR, 256), row)],
        out_shape=[_f32(C, 256)],
    )(F, ATR16, Wg1, Wg2)[0]


def kernel(x, pos, edge_index, edge_attr, batch, W_conv, W_gather):
    f32 = jnp.float32
    src, dst = edge_index[0], edge_index[1]

    xs = x[src]
    m20 = jnp.concatenate([xs, edge_attr], axis=1) @ W_conv[:, :20]
    O20m = jax.ops.segment_sum(m20, dst, num_segments=N)
    A = jax.ops.segment_sum(xs, dst, num_segments=N)
    EA = jax.ops.segment_sum(edge_attr, dst, num_segments=N)
    O20 = jnp.pad(O20m, ((0, 0), (0, 12)))
    EA16 = jnp.pad(EA, ((0, 0), (0, 12)))

    pos8 = jnp.pad(pos, ((0, 0), (0, 13)))
    Wf1 = W_conv[:D, 20:]
    Wf2 = jnp.pad(W_conv[D:, 20:], ((0, 12), (0, 0)))
    Wg1 = W_gather[:D, :]
    Wg2 = jnp.pad(W_gather[D:, :], ((0, 13), (0, 0)))
    feat, xn2, bp16, msc = _tc_feat(A, EA16, O20, pos8, Wf1, Wf2, Wg1, Wg2)

    bp = bp16[:, :12]
    bloom_pos = bp.reshape(N * K, 3)
    cell = jnp.floor(bloom_pos / MINR).astype(jnp.int32)
    b_pt = jnp.repeat(batch, K)
    h = ((cell[:, 0] * 73856093) ^ (cell[:, 1] * 19349663)
         ^ (cell[:, 2] * 83492791) ^ (b_pt * 2654435))
    cid = jnp.mod(h, C)
    onehot = (b_pt[:, None]
              == jnp.arange(NB, dtype=jnp.int32)[None, :]).astype(f32)
    srows = jnp.concatenate([bloom_pos, jnp.ones((N * K, 1), f32), onehot],
                            axis=1)
    S = jax.ops.segment_sum(srows, cid, num_segments=C)
    cnt = S[:, 3:4]
    npc3 = S[:, :3] / jnp.clip(cnt, 1.0)
    npc128 = jnp.pad(npc3, ((0, 0), (0, 125)))
    batc = jnp.full((C,), jnp.iinfo(jnp.int32).min, jnp.int32)
    for b in range(NB):
        batc = jnp.where(S[:, 4 + b] > 0, b, batc)

    cid2 = cid.reshape(N, K)
    F = jax.ops.segment_sum(feat, cid2[:, 0], num_segments=C)
    for k in range(1, K):
        F = F + jax.ops.segment_sum(feat, cid2[:, k], num_segments=C)
    gat = jnp.repeat(pos, K, axis=0) - npc3[cid]
    ATR = jax.ops.segment_sum(gat, cid, num_segments=C)
    ATR16 = jnp.pad(ATR, ((0, 0), (0, 13)))

    xn1 = _tc_out(F, ATR16, Wg1, Wg2)

    srcp = jnp.pad(src, (0, EP - E))
    dstp = jnp.pad(dst, (0, EP - E), constant_values=-1)
    rep = jnp.pad(cid2[:, 0], (0, NP - N))
    nso, ndo, ner = sc_edge_remap(rep, srcp, dstp, npc128)

    x_new = jnp.concatenate([xn1, xn2], axis=0)
    new_pos = jnp.concatenate([npc3, msc[:, :3]], axis=0)
    new_edge_index = jnp.stack([nso[:E], ndo[:E]])
    new_edge_attr = ner[:E, :3]
    new_batch = jnp.concatenate([batc, batch])
    return x_new, new_pos, new_edge_index, new_edge_attr, new_batch


# speedup vs baseline: 2.0605x; 1.0843x over previous
import functools

import jax
import jax.numpy as jnp
from jax import lax
from jax.experimental import pallas as pl
from jax.experimental.pallas import tpu as pltpu
from jax.experimental.pallas import tpu_sc as plsc

N = 10000
E = 160000
D = 256
K = 4
C = 16384
MINR = 0.1
NB = 4

NP = 10240
EP = 163840

HI = lax.Precision.HIGHEST

_MESH = plsc.VectorSubcoreMesh(core_axis_name="c", subcore_axis_name="s")


def _f32(*shape):
    return jax.ShapeDtypeStruct(shape, jnp.float32)


def _i32(*shape):
    return jax.ShapeDtypeStruct(shape, jnp.int32)


@functools.partial(
    pl.kernel,
    out_type=[_i32(EP), _i32(EP), _f32(EP, 16)],
    mesh=_MESH,
    scratch_types=[
        pltpu.VMEM((128,), jnp.int32),
        pltpu.VMEM((128,), jnp.int32),
        pltpu.VMEM((128,), jnp.int32),
        pltpu.VMEM((128,), jnp.int32),
        pltpu.VMEM((128, 128), jnp.float32),
        pltpu.VMEM((128, 128), jnp.float32),
        pltpu.VMEM((128, 16), jnp.float32),
    ],
)
def sc_edge_remap(rep_hbm, src_hbm, dst_hbm, npc_hbm,
                  ns_out, nd_out, attr_out,
                  srcb, dstb, nsb, ndb, bs, bd, bo):
    c = lax.axis_index("c")
    s = lax.axis_index("s")
    w = s * 2 + c

    def chunk(t, _):
        base = (w * 40 + t) * 128
        pltpu.sync_copy(src_hbm.at[pl.ds(base, 128)], srcb)
        pltpu.sync_copy(dst_hbm.at[pl.ds(base, 128)], dstb)
        for g in range(8):
            srcb[pl.ds(g * 16, 16)] = jnp.maximum(srcb[pl.ds(g * 16, 16)], 0)
            dstb[pl.ds(g * 16, 16)] = jnp.maximum(dstb[pl.ds(g * 16, 16)], 0)
        pltpu.sync_copy(rep_hbm.at[srcb], nsb)
        pltpu.sync_copy(rep_hbm.at[dstb], ndb)
        pltpu.sync_copy(nsb, ns_out.at[pl.ds(base, 128)])
        pltpu.sync_copy(ndb, nd_out.at[pl.ds(base, 128)])
        pltpu.sync_copy(npc_hbm.at[nsb], bs)
        pltpu.sync_copy(npc_hbm.at[ndb], bd)
        for i in range(128):
            bo[i] = bd[i, pl.ds(0, 16)] - bs[i, pl.ds(0, 16)]
        pltpu.sync_copy(bo, attr_out.at[pl.ds(base, 128)])
        return 0

    lax.fori_loop(0, 40, chunk, 0)


def _tc_feat_body(a_ref, ea_ref, o20_ref, pos_ref, wf1, wf2, wg1, wg2,
                  feat_o, xn2, bp, msc):
    feat = (jnp.dot(a_ref[...], wf1[...], precision=HI)
            + jnp.dot(ea_ref[...], wf2[...], precision=HI))
    o20 = o20_ref[...]
    pos3 = pos_ref[...][:, :3]
    n = o20.shape[0]
    sph12 = o20[:, :12]
    maskv = (o20[:, 16:17] > 0.5).astype(jnp.float32)
    dispv = jnp.concatenate([o20[:, 19:20], o20[:, 17:18], o20[:, 18:19]],
                            axis=1)
    cpos = pos3 + dispv
    post = jnp.concatenate([pos3, pos3, pos3, pos3], axis=1)
    bp12 = post + jnp.tanh(sph12) * (2.0 * MINR)
    nd16 = jnp.concatenate([-dispv, jnp.zeros((n, 13), jnp.float32)], axis=1)
    xn2v = (jnp.dot(feat, wg1[...], precision=HI)
            + jnp.dot(nd16, wg2[...], precision=HI)) * maskv
    feat_o[...] = feat
    xn2[...] = xn2v
    bp[...] = jnp.concatenate([bp12, jnp.zeros((n, 4), jnp.float32)], axis=1)
    msc[...] = jnp.concatenate([cpos, -dispv, maskv,
                                jnp.zeros((n, 9), jnp.float32)], axis=1)


def _tc_feat(A, EA16, O20, pos8, Wf1, Wf2, Wg1, Wg2):
    BR = 1000
    grid = (N // BR,)
    row = lambda i: (i, 0)
    full = lambda i: (0, 0)
    return pl.pallas_call(
        _tc_feat_body,
        grid=grid,
        in_specs=[
            pl.BlockSpec((BR, 256), row),
            pl.BlockSpec((BR, 16), row),
            pl.BlockSpec((BR, 32), row),
            pl.BlockSpec((BR, 16), row),
            pl.BlockSpec((256, 256), full),
            pl.BlockSpec((16, 256), full),
            pl.BlockSpec((256, 256), full),
            pl.BlockSpec((16, 256), full),
        ],
        out_specs=[
            pl.BlockSpec((BR, 256), row),
            pl.BlockSpec((BR, 256), row),
            pl.BlockSpec((BR, 16), row),
            pl.BlockSpec((BR, 16), row),
        ],
        out_shape=[_f32(N, 256), _f32(N, 256), _f32(N, 16), _f32(N, 16)],
    )(A, EA16, O20, pos8, Wf1, Wf2, Wg1, Wg2)


def _tc_out_body(F, atr, wg1, wg2, out):
    out[...] = (jnp.dot(F[...], wg1[...], precision=HI)
                + jnp.dot(atr[...], wg2[...], precision=HI))


def _tc_out(F, ATR16, Wg1, Wg2):
    BR = 1024
    grid = (C // BR,)
    row = lambda i: (i, 0)
    full = lambda i: (0, 0)
    return pl.pallas_call(
        _tc_out_body,
        grid=grid,
        in_specs=[
            pl.BlockSpec((BR, 256), row),
            pl.BlockSpec((BR, 16), row),
            pl.BlockSpec((256, 256), full),
            pl.BlockSpec((16, 256), full),
        ],
        out_specs=[pl.BlockSpec((BR, 256), row)],
        out_shape=[_f32(C, 256)],
    )(F, ATR16, Wg1, Wg2)[0]


def kernel(x, pos, edge_index, edge_attr, batch, W_conv, W_gather):
    f32 = jnp.float32
    src, dst = edge_index[0], edge_index[1]

    xs = x[src]
    m20 = jnp.concatenate([xs, edge_attr], axis=1) @ W_conv[:, :20]
    seg = jax.ops.segment_sum(
        jnp.concatenate([m20, xs, edge_attr], axis=1), dst, num_segments=N)
    O20m, A, EA = seg[:, :20], seg[:, 20:20 + D], seg[:, 20 + D:]
    O20 = jnp.pad(O20m, ((0, 0), (0, 12)))
    EA16 = jnp.pad(EA, ((0, 0), (0, 12)))

    pos8 = jnp.pad(pos, ((0, 0), (0, 13)))
    Wf1 = W_conv[:D, 20:]
    Wf2 = jnp.pad(W_conv[D:, 20:], ((0, 12), (0, 0)))
    Wg1 = W_gather[:D, :]
    Wg2 = jnp.pad(W_gather[D:, :], ((0, 13), (0, 0)))
    feat, xn2, bp16, msc = _tc_feat(A, EA16, O20, pos8, Wf1, Wf2, Wg1, Wg2)

    bp = bp16[:, :12]
    bloom_pos = bp.reshape(N * K, 3)
    cell = jnp.floor(bloom_pos / MINR).astype(jnp.int32)
    b_pt = jnp.repeat(batch, K)
    h = ((cell[:, 0] * 73856093) ^ (cell[:, 1] * 19349663)
         ^ (cell[:, 2] * 83492791) ^ (b_pt * 2654435))
    cid = jnp.mod(h, C)
    onehot = (b_pt[:, None]
              == jnp.arange(NB, dtype=jnp.int32)[None, :]).astype(f32)
    srows = jnp.concatenate([bloom_pos, jnp.ones((N * K, 1), f32), onehot],
                            axis=1)
    S = jax.ops.segment_sum(srows, cid, num_segments=C)
    cnt = S[:, 3:4]
    npc3 = S[:, :3] / jnp.clip(cnt, 1.0)
    npc128 = jnp.pad(npc3, ((0, 0), (0, 125)))
    batc = jnp.full((C,), jnp.iinfo(jnp.int32).min, jnp.int32)
    for b in range(NB):
        batc = jnp.where(S[:, 4 + b] > 0, b, batc)

    cid2 = cid.reshape(N, K)
    F = jax.ops.segment_sum(jnp.repeat(feat, K, axis=0), cid,
                            num_segments=C)
    gat = jnp.repeat(pos, K, axis=0) - npc3[cid]
    ATR = jax.ops.segment_sum(gat, cid, num_segments=C)
    ATR16 = jnp.pad(ATR, ((0, 0), (0, 13)))

    xn1 = _tc_out(F, ATR16, Wg1, Wg2)

    srcp = jnp.pad(src, (0, EP - E))
    dstp = jnp.pad(dst, (0, EP - E), constant_values=-1)
    rep = jnp.pad(cid2[:, 0], (0, NP - N))
    nso, ndo, ner = sc_edge_remap(rep, srcp, dstp, npc128)

    x_new = jnp.concatenate([xn1, xn2], axis=0)
    new_pos = jnp.concatenate([npc3, msc[:, :3]], axis=0)
    new_edge_index = jnp.stack([nso[:E], ndo[:E]])
    new_edge_attr = ner[:E, :3]
    new_batch = jnp.concatenate([batc, batch])
    return x_new, new_pos, new_edge_index, new_edge_attr, new_batch
